# Initial kernel scaffold; baseline (speedup 1.0000x reference)
import jax
import jax.numpy as jnp
from jax.experimental import pallas as pl


def kernel(input_ids, trigram_w, fourgram_w):
    raise NotImplementedError("write your pallas kernel here")


# speedup vs baseline: 1.8631x; 1.8631x over previous
import functools

import jax
import jax.numpy as jnp
from jax import lax
from jax.experimental import pallas as pl
from jax.experimental.pallas import tpu as pltpu
from jax.experimental.pallas import tpu_sc as plsc

HASH_BUCKETS = 1000000
DIM = 32
B, L = 4096, 200
NC, NS = 2, 16
NW = NC * NS
ROWS_PER_W = B // NW
POS_PER_W = ROWS_PER_W * L
TOK0 = 8
CHUNK = 128
NCHUNK = POS_PER_W // CHUNK
NVEC = L // 16


def _i32(v):
    return jnp.int32(v)


def _hash16(a, b, c, d):
    k257, k9973 = _i32(257), _i32(9973)
    kq, ks, k1000 = _i32(33555), _i32(25), _i32(1000)
    k537000, k65537 = _i32(537000), _i32(65537)
    qd = (d * kq) >> ks
    rd = d - qd * k1000
    qc = (c * kq) >> ks
    rc = c - qc * k1000
    tri = b + c * k257 + qd * k537000 + rd * k65537
    four = a + b * k257 + qc * k537000 + rc * k65537 + d * k9973
    for k in range(7, -1, -1):
        cst = _i32(1000000 << k)
        tri = jnp.where(tri >= cst, tri - cst, tri)
    for k in range(9, -1, -1):
        cst = _i32(1000000 << k)
        four = jnp.where(four >= cst, four - cst, four)
    return tri, four


_mesh = plsc.VectorSubcoreMesh(core_axis_name="c", subcore_axis_name="s")


@functools.partial(
    pl.kernel,
    out_type=jax.ShapeDtypeStruct((B * L, DIM), jnp.float32),
    mesh=_mesh,
    compiler_params=pltpu.CompilerParams(use_tc_tiling_on_sc=False),
    scratch_types=[
        pltpu.VMEM((TOK0 + POS_PER_W,), jnp.int32),
        pltpu.VMEM((POS_PER_W,), jnp.int32),
        pltpu.VMEM((POS_PER_W,), jnp.int32),
        pltpu.VMEM((CHUNK, DIM), jnp.float32),
        pltpu.VMEM((CHUNK, DIM), jnp.float32),
        pltpu.SemaphoreType.DMA,
        pltpu.SemaphoreType.DMA,
    ],
)
def _embed(ids_hbm, tri_hbm, four_hbm, out_hbm,
           tok, itri, ifour, rtri, rfour, sem1, sem2):
    wid = (lax.axis_index("s").astype(jnp.int32) * _i32(NC)
           + lax.axis_index("c").astype(jnp.int32))
    base = wid * _i32(POS_PER_W)

    tok[pl.ds(0, 16)] = jnp.zeros((16,), jnp.int32)
    pltpu.sync_copy(ids_hbm.at[pl.ds(base, POS_PER_W)],
                    tok.at[pl.ds(TOK0, POS_PER_W)])

    iota16 = lax.iota(jnp.int32, 16)

    def taps(p):
        d = tok[pl.ds(p, 16)]
        c = tok[pl.ds(p - _i32(1), 16)]
        b = tok[pl.ds(p - _i32(2), 16)]
        a = tok[pl.ds(p - _i32(3), 16)]
        return a, b, c, d

    def hash_row(r, carry):
        rb = r * _i32(L)
        a, b, c, d = taps(rb + _i32(TOK0))
        zero = jnp.zeros((16,), jnp.int32)
        a = jnp.where(iota16 >= _i32(3), a, zero)
        b = jnp.where(iota16 >= _i32(2), b, zero)
        c = jnp.where(iota16 >= _i32(1), c, zero)
        tri, four = _hash16(a, b, c, d)
        itri[pl.ds(rb, 16)] = tri
        ifour[pl.ds(rb, 16)] = four

        def body(k, carry2):
            off = rb + k * _i32(16)
            a, b, c, d = taps(off + _i32(TOK0))
            tri, four = _hash16(a, b, c, d)
            itri[pl.ds(off, 16)] = tri
            ifour[pl.ds(off, 16)] = four
            return carry2

        lax.fori_loop(_i32(1), _i32(NVEC), body, 0)
        off = rb + _i32(L - 16)
        a, b, c, d = taps(off + _i32(TOK0))
        tri, four = _hash16(a, b, c, d)
        itri[pl.ds(off, 16)] = tri
        ifour[pl.ds(off, 16)] = four
        return carry

    lax.fori_loop(_i32(0), _i32(ROWS_PER_W), hash_row, 0)

    def gather_chunk(j, carry):
        fb = j * _i32(CHUNK)
        cp1 = pltpu.async_copy(tri_hbm.at[itri.at[pl.ds(fb, CHUNK)]],
                               rtri, sem1)
        cp2 = pltpu.async_copy(four_hbm.at[ifour.at[pl.ds(fb, CHUNK)]],
                               rfour, sem2)
        cp1.wait()
        cp2.wait()

        def add_body(i, carry2):
            for h in range(2):
                sl = pl.ds(h * 16, 16)
                rtri[i, sl] = rtri[i, sl] + rfour[i, sl]
            return carry2

        lax.fori_loop(_i32(0), _i32(CHUNK), add_body, 0)

        pltpu.sync_copy(rtri, out_hbm.at[pl.ds(base + fb, CHUNK)])
        return carry

    lax.fori_loop(_i32(0), _i32(NCHUNK), gather_chunk, 0)


def kernel(input_ids, trigram_w, fourgram_w):
    ids = input_ids.reshape(-1).astype(jnp.int32)
    out = _embed(ids, trigram_w, fourgram_w)
    return out.reshape(B, L, DIM)


# speedup vs baseline: 2.1783x; 1.1692x over previous
import functools

import jax
import jax.numpy as jnp
from jax import lax
from jax.experimental import pallas as pl
from jax.experimental.pallas import tpu as pltpu
from jax.experimental.pallas import tpu_sc as plsc

HASH_BUCKETS = 1000000
DIM = 32
B, L = 4096, 200
NC, NS = 2, 16
NW = NC * NS
ROWS_PER_W = B // NW
POS_PER_W = ROWS_PER_W * L
TOK0 = 8
CHUNK = 128
NCHUNK = POS_PER_W // CHUNK
NVEC = L // 16
NBUF = 4
NGRP = NCHUNK // NBUF


def _i32(v):
    return jnp.int32(v)


def _hash16(a, b, c, d):
    k257, k9973 = _i32(257), _i32(9973)
    kq, ks, k1000 = _i32(33555), _i32(25), _i32(1000)
    k537000, k65537 = _i32(537000), _i32(65537)
    qd = (d * kq) >> ks
    rd = d - qd * k1000
    qc = (c * kq) >> ks
    rc = c - qc * k1000
    tri = b + c * k257 + qd * k537000 + rd * k65537
    four = a + b * k257 + qc * k537000 + rc * k65537 + d * k9973
    for k in range(7, -1, -1):
        cst = _i32(1000000 << k)
        tri = jnp.where(tri >= cst, tri - cst, tri)
    for k in range(9, -1, -1):
        cst = _i32(1000000 << k)
        four = jnp.where(four >= cst, four - cst, four)
    return tri, four


_mesh = plsc.VectorSubcoreMesh(core_axis_name="c", subcore_axis_name="s")


@functools.partial(
    pl.kernel,
    out_type=jax.ShapeDtypeStruct((B * L, DIM), jnp.float32),
    mesh=_mesh,
    compiler_params=pltpu.CompilerParams(use_tc_tiling_on_sc=False),
    scratch_types=[
        pltpu.VMEM((TOK0 + POS_PER_W,), jnp.int32),
        pltpu.VMEM((POS_PER_W,), jnp.int32),
        pltpu.VMEM((POS_PER_W,), jnp.int32),
        pltpu.VMEM((NBUF, CHUNK, DIM), jnp.float32),
        pltpu.VMEM((NBUF, CHUNK, DIM), jnp.float32),
        pltpu.VMEM((NBUF, CHUNK, DIM), jnp.float32),
    ] + [pltpu.SemaphoreType.DMA] * (3 * NBUF),
)
def _embed(ids_hbm, tri_hbm, four_hbm, out_hbm,
           tok, itri, ifour, rtri, rfour, obuf, *sems):
    tsem = sems[0:NBUF]
    fsem = sems[NBUF:2 * NBUF]
    osem = sems[2 * NBUF:3 * NBUF]
    wid = (lax.axis_index("s").astype(jnp.int32) * _i32(NC)
           + lax.axis_index("c").astype(jnp.int32))
    base = wid * _i32(POS_PER_W)

    tok[pl.ds(0, 16)] = jnp.zeros((16,), jnp.int32)
    pltpu.sync_copy(ids_hbm.at[pl.ds(base, POS_PER_W)],
                    tok.at[pl.ds(TOK0, POS_PER_W)])

    iota16 = lax.iota(jnp.int32, 16)

    def taps(p):
        d = tok[pl.ds(p, 16)]
        c = tok[pl.ds(p - _i32(1), 16)]
        b = tok[pl.ds(p - _i32(2), 16)]
        a = tok[pl.ds(p - _i32(3), 16)]
        return a, b, c, d

    def hash_row(r, carry):
        rb = r * _i32(L)
        a, b, c, d = taps(rb + _i32(TOK0))
        zero = jnp.zeros((16,), jnp.int32)
        a = jnp.where(iota16 >= _i32(3), a, zero)
        b = jnp.where(iota16 >= _i32(2), b, zero)
        c = jnp.where(iota16 >= _i32(1), c, zero)
        tri, four = _hash16(a, b, c, d)
        itri[pl.ds(rb, 16)] = tri
        ifour[pl.ds(rb, 16)] = four

        def body(k, carry2):
            off = rb + k * _i32(16)
            a, b, c, d = taps(off + _i32(TOK0))
            tri, four = _hash16(a, b, c, d)
            itri[pl.ds(off, 16)] = tri
            ifour[pl.ds(off, 16)] = four
            return carry2

        lax.fori_loop(_i32(1), _i32(NVEC), body, 0)
        off = rb + _i32(L - 16)
        a, b, c, d = taps(off + _i32(TOK0))
        tri, four = _hash16(a, b, c, d)
        itri[pl.ds(off, 16)] = tri
        ifour[pl.ds(off, 16)] = four
        return carry

    lax.fori_loop(_i32(0), _i32(ROWS_PER_W), hash_row, 0)

    def fire_gathers(fb, b):
        pltpu.async_copy(tri_hbm.at[itri.at[pl.ds(fb, CHUNK)]],
                         rtri.at[_i32(b)], tsem[b])
        pltpu.async_copy(four_hbm.at[ifour.at[pl.ds(fb, CHUNK)]],
                         rfour.at[_i32(b)], fsem[b])

    def wait_gathers(fb, b):
        pltpu.make_async_copy(tri_hbm.at[itri.at[pl.ds(fb, CHUNK)]],
                              rtri.at[_i32(b)], tsem[b]).wait()
        pltpu.make_async_copy(four_hbm.at[ifour.at[pl.ds(fb, CHUNK)]],
                              rfour.at[_i32(b)], fsem[b]).wait()

    def drain_out(fb, b):
        pltpu.make_async_copy(obuf.at[_i32(b)],
                              out_hbm.at[pl.ds(base + fb, CHUNK)],
                              osem[b]).wait()

    for b in range(NBUF):
        fire_gathers(_i32(b * CHUNK), b)

    def group(g, carry):
        j0 = g * _i32(NBUF)
        for b in range(NBUF):
            jj = j0 + _i32(b)
            fb = jj * _i32(CHUNK)
            wait_gathers(fb, b)

            @pl.when(g > _i32(0))
            def _():
                drain_out(fb, b)

            def add_body(i, carry2):
                i8 = i * _i32(8)
                for u in range(8):
                    row = i8 + _i32(u)
                    for h in range(2):
                        sl = pl.ds(h * 16, 16)
                        obuf[_i32(b), row, sl] = rtri[_i32(b), row, sl] + rfour[_i32(b), row, sl]
                return carry2

            lax.fori_loop(_i32(0), _i32(CHUNK // 8), add_body, 0)
            pltpu.async_copy(obuf.at[_i32(b)],
                             out_hbm.at[pl.ds(base + fb, CHUNK)], osem[b])

            @pl.when(jj + _i32(NBUF) < _i32(NCHUNK))
            def _():
                fire_gathers(fb + _i32(NBUF * CHUNK), b)

        return carry

    lax.fori_loop(_i32(0), _i32(NGRP), group, 0)
    for b in range(NBUF):
        drain_out(_i32((NGRP - 1) * NBUF + b) * _i32(CHUNK), b)


def kernel(input_ids, trigram_w, fourgram_w):
    ids = input_ids.reshape(-1).astype(jnp.int32)
    out = _embed(ids, trigram_w, fourgram_w)
    return out.reshape(B, L, DIM)
